# baseline (speedup 1.0000x reference)
import functools

import jax
import jax.numpy as jnp
import numpy as np
from jax import lax
from jax.experimental import pallas as pl
from jax.experimental.pallas import tpu as pltpu
from jax.experimental.pallas import tpu_sc as plsc

V = 100000
B = 1024
_LS = 0.1
_S = np.float32(_LS / (V - 2))
_C = np.float32(1.0 - _LS)
_KL_CONST = np.float32((V - 1) * (_S * np.float32(np.log(_S))) + _C * np.float32(np.log(_C)))
_CMS = np.float32(_C - _S)

_NC = 2
_NS = 16
_L = 16
_NW = _NC * _NS
_BPW = B // _NW

_sc_mesh = plsc.VectorSubcoreMesh(core_axis_name="c", subcore_axis_name="s")


@functools.partial(
    pl.kernel,
    mesh=_sc_mesh,
    out_type=jax.ShapeDtypeStruct((B,), jnp.float32),
    scratch_types=[
        pltpu.VMEM((_BPW,), jnp.int32),
        pltpu.VMEM((_BPW,), jnp.int32),
        pltpu.VMEM((_BPW,), jnp.float32),
        pltpu.SemaphoreType.DMA,
    ],
)
def _sc_gather(flat_hbm, tgt_hbm, out_hbm, tgt_v, idx_v, vals_v, sem):
    wid = lax.axis_index("s") * _NC + lax.axis_index("c")
    base = wid * _BPW
    pltpu.sync_copy(tgt_hbm.at[pl.ds(base, _BPW)], tgt_v)
    for j in range(_BPW // _L):
        row = base + j * _L + lax.iota(jnp.int32, _L)
        t = tgt_v[pl.ds(j * _L, _L)]
        idx_v[pl.ds(j * _L, _L)] = t + row * V
    pltpu.async_copy(flat_hbm.at[idx_v], vals_v, sem).wait()
    pltpu.sync_copy(vals_v, out_hbm.at[pl.ds(base, _BPW)])


_BB = 16
_NB = B // _BB


def _tc_body(x_ref, v_ref, o_ref):
    rs = jnp.sum(x_ref[...], axis=1)
    o_ref[0, 0, :] = _KL_CONST - _S * rs - _CMS * v_ref[0, 0, :]


_tc_call = pl.pallas_call(
    _tc_body,
    grid=(_NB,),
    in_specs=[
        pl.BlockSpec((_BB, V), lambda i: (i, 0)),
        pl.BlockSpec((1, 1, _BB), lambda i: (i, 0, 0)),
    ],
    out_specs=pl.BlockSpec((1, 1, _BB), lambda i: (i, 0, 0)),
    out_shape=jax.ShapeDtypeStruct((_NB, 1, _BB), jnp.float32),
)


def kernel(output, target):
    tgt = target.astype(jnp.int32)
    flat = output.reshape(-1)
    vals = _sc_gather(flat, tgt)
    res = _tc_call(output, vals.reshape(_NB, 1, _BB))
    return res.reshape(B)


# speedup vs baseline: 1.0100x; 1.0100x over previous
import functools

import jax
import jax.numpy as jnp
import numpy as np
from jax import lax
from jax.experimental import pallas as pl
from jax.experimental.pallas import tpu as pltpu
from jax.experimental.pallas import tpu_sc as plsc

V = 100000
B = 1024
_LS = 0.1
_S = np.float32(_LS / (V - 2))
_C = np.float32(1.0 - _LS)
_KL_CONST = np.float32((V - 1) * (_S * np.float32(np.log(_S))) + _C * np.float32(np.log(_C)))
_CMS = np.float32(_C - _S)

_NC = 2
_NS = 16
_L = 16
_NW = _NC * _NS
_BPW = B // _NW

_sc_mesh = plsc.VectorSubcoreMesh(core_axis_name="c", subcore_axis_name="s")


@functools.partial(
    pl.kernel,
    mesh=_sc_mesh,
    out_type=jax.ShapeDtypeStruct((B,), jnp.float32),
    scratch_types=[
        pltpu.VMEM((_BPW,), jnp.int32),
        pltpu.VMEM((_BPW,), jnp.int32),
        pltpu.VMEM((_BPW,), jnp.float32),
        pltpu.SemaphoreType.DMA,
    ],
)
def _sc_gather(flat_hbm, tgt_hbm, out_hbm, tgt_v, idx_v, vals_v, sem):
    wid = lax.axis_index("s") * _NC + lax.axis_index("c")
    base = wid * _BPW
    pltpu.sync_copy(tgt_hbm.at[pl.ds(base, _BPW)], tgt_v)
    for j in range(_BPW // _L):
        row = base + j * _L + lax.iota(jnp.int32, _L)
        t = tgt_v[pl.ds(j * _L, _L)]
        idx_v[pl.ds(j * _L, _L)] = t + row * V
    pltpu.async_copy(flat_hbm.at[idx_v], vals_v, sem).wait()
    pltpu.sync_copy(vals_v, out_hbm.at[pl.ds(base, _BPW)])


_RB = 8
_K = 8
_NCHUNK = B // _RB
_NROUND = _NCHUNK // _K


def _tc_body(x_hbm, v_ref, o_ref, buf, sem):
    def copy(c, b):
        return pltpu.make_async_copy(
            x_hbm.at[pl.ds(c * _RB, _RB)], buf.at[b], sem.at[b]
        )

    for b in range(_K):
        copy(b, b).start()

    def round_body(r, carry):
        for b in range(_K):
            c = r * _K + b
            copy(c, b).wait()
            rs = jnp.sum(buf[b], axis=1, keepdims=True)
            o_ref[pl.ds(c * _RB, _RB), :] = (
                _KL_CONST - _S * rs - _CMS * v_ref[pl.ds(c * _RB, _RB), :]
            )

            @pl.when(r + 1 < _NROUND)
            def _():
                copy(c + _K, b).start()

        return carry

    lax.fori_loop(0, _NROUND, round_body, 0)


_tc_call = pl.pallas_call(
    _tc_body,
    in_specs=[
        pl.BlockSpec(memory_space=pl.ANY),
        pl.BlockSpec(memory_space=pltpu.VMEM),
    ],
    out_specs=pl.BlockSpec(memory_space=pltpu.VMEM),
    out_shape=jax.ShapeDtypeStruct((B, 1), jnp.float32),
    scratch_shapes=[
        pltpu.VMEM((_K, _RB, V), jnp.float32),
        pltpu.SemaphoreType.DMA((_K,)),
    ],
)


def kernel(output, target):
    tgt = target.astype(jnp.int32)
    flat = output.reshape(-1)
    vals = _sc_gather(flat, tgt)
    res = _tc_call(output, vals.reshape(B, 1))
    return res.reshape(B)
